# Initial kernel scaffold; baseline (speedup 1.0000x reference)
import jax
import jax.numpy as jnp
from jax.experimental import pallas as pl


def kernel(user_features, movie_features, user_table, movie_table):
    raise NotImplementedError("write your pallas kernel here")


# speedup vs baseline: 1.2070x; 1.2070x over previous
import functools

import jax
import jax.numpy as jnp
from jax import lax
from jax.experimental import pallas as pl
from jax.experimental.pallas import tpu as pltpu
from jax.experimental.pallas import tpu_sc as plsc

AGE_CATEGORIES = 7
OCC_CATEGORIES = 21
GENRES_CATEGORIES = 18
EMBED_DIM = 128
BATCH = 16384

NUM_CORES = 2
NUM_SUBCORES = 16
NW = NUM_CORES * NUM_SUBCORES
BPW = BATCH // NW
CHUNK = 128
NCH = BPW // CHUNK

_MESH = plsc.VectorSubcoreMesh(core_axis_name="c", subcore_axis_name="s")


@functools.partial(
    pl.kernel,
    out_type=(
        jax.ShapeDtypeStruct((BATCH, EMBED_DIM), jnp.float32),
        jax.ShapeDtypeStruct((BATCH, EMBED_DIM), jnp.float32),
    ),
    mesh=_MESH,
    scratch_types=[
        pltpu.VMEM((NCH, CHUNK), jnp.int32),
        pltpu.VMEM((BPW, EMBED_DIM), jnp.float32),
        pltpu.SemaphoreType.DMA,
    ],
)
def _two_tower(ut, mt, uidx, midx, uout, mout, idx_v, rows_v, sem):
    wid = lax.axis_index("s") * NUM_CORES + lax.axis_index("c")
    base = wid * BPW

    pltpu.sync_copy(uidx.at[wid], idx_v)
    copies = [
        pltpu.async_copy(ut.at[idx_v.at[j]],
                         rows_v.at[pl.ds(j * CHUNK, CHUNK)], sem)
        for j in range(NCH)
    ]
    for c in copies:
        c.wait()
    pltpu.sync_copy(rows_v, uout.at[pl.ds(base, BPW)])

    pltpu.sync_copy(midx.at[wid], idx_v)
    copies = [
        pltpu.async_copy(mt.at[idx_v.at[j]],
                         rows_v.at[pl.ds(j * CHUNK, CHUNK)], sem)
        for j in range(NCH)
    ]
    for c in copies:
        c.wait()
    pltpu.sync_copy(rows_v, mout.at[pl.ds(base, BPW)])


def kernel(user_features, movie_features, user_table, movie_table):
    uidx = user_features[:, 0].astype(jnp.int32).reshape(NW, NCH, CHUNK)
    midx = movie_features[:, 0].astype(jnp.int32).reshape(NW, NCH, CHUNK)
    uemb, memb = _two_tower(user_table, movie_table, uidx, midx)
    user_embedded = jnp.concatenate([uemb, user_features[:, 1:]], axis=1)
    movie_embedded = jnp.concatenate([memb, movie_features[:, 1:]], axis=1)
    return (user_embedded, movie_embedded)
